# baseline (speedup 1.0000x reference)
import functools

import jax
import jax.numpy as jnp
from jax import lax
from jax.experimental import pallas as pl
from jax.experimental.pallas import tpu as pltpu
from jax.experimental.pallas import tpu_sc as plsc

N_NODE = 10000
E = 320000
D = 128
NC = 2
NS = 16
NT = NC * NS
EPT = E // NT
BLK = 80
NBLK = EPT // BLK
GRP = BLK // 16
SPAD = 10240
NEG = -3.0e38

_mesh = plsc.VectorSubcoreMesh(core_axis_name="c", subcore_axis_name="s")


def _pack_pairs(yev, yod):
    ev = jax.lax.bitcast_convert_type(yev.astype(jnp.bfloat16), jnp.uint16)
    od = jax.lax.bitcast_convert_type(yod.astype(jnp.bfloat16), jnp.uint16)
    word = ev.astype(jnp.uint32) | (od.astype(jnp.uint32) << 16)
    return jax.lax.bitcast_convert_type(word, jnp.float32)


def _proj2_body(x_ref, w1e_ref, b1e_ref, w1o_ref, b1o_ref,
                w2e_ref, b2e_ref, w2o_ref, b2o_ref, o1_ref, o2_ref):
    x = x_ref[...]
    o1_ref[...] = _pack_pairs(
        jnp.dot(x, w1e_ref[...], preferred_element_type=jnp.float32) + b1e_ref[...],
        jnp.dot(x, w1o_ref[...], preferred_element_type=jnp.float32) + b1o_ref[...],
    )
    o2_ref[...] = _pack_pairs(
        jnp.dot(x, w2e_ref[...], preferred_element_type=jnp.float32) + b2e_ref[...],
        jnp.dot(x, w2o_ref[...], preferred_element_type=jnp.float32) + b2o_ref[...],
    )


DH = D // 2


def _eproj_body(xe_ref, xo_ref, we_ref, be_ref, wo_ref, bo_ref, o_ref):
    xe = xe_ref[...]
    xo = xo_ref[...]
    pe = _pack_pairs(
        jnp.dot(xe, we_ref[...], preferred_element_type=jnp.float32) + be_ref[...],
        jnp.dot(xe, wo_ref[...], preferred_element_type=jnp.float32) + bo_ref[...],
    )
    po = _pack_pairs(
        jnp.dot(xo, we_ref[...], preferred_element_type=jnp.float32) + be_ref[...],
        jnp.dot(xo, wo_ref[...], preferred_element_type=jnp.float32) + bo_ref[...],
    )
    o_ref[...] = jnp.concatenate([pe, po], axis=1)


def _eproj(xe, xo, w, b, blk):
    n2 = xe.shape[0]
    return pl.pallas_call(
        _eproj_body,
        grid=(n2 // blk,),
        in_specs=[
            pl.BlockSpec((blk, 16), lambda i: (i, 0)),
            pl.BlockSpec((blk, 16), lambda i: (i, 0)),
            pl.BlockSpec((16, DH), lambda i: (0, 0)),
            pl.BlockSpec((1, DH), lambda i: (0, 0)),
            pl.BlockSpec((16, DH), lambda i: (0, 0)),
            pl.BlockSpec((1, DH), lambda i: (0, 0)),
        ],
        out_specs=pl.BlockSpec((blk, D), lambda i: (i, 0)),
        out_shape=jax.ShapeDtypeStruct((n2, D), jnp.float32),
    )(xe, xo, w[:, 0::2], b[0::2].reshape(1, DH), w[:, 1::2], b[1::2].reshape(1, DH))


def _proj2(x, w1, b1, w2, b2, blk):
    n, k = x.shape
    grid = n // blk
    return pl.pallas_call(
        _proj2_body,
        grid=(grid,),
        in_specs=[pl.BlockSpec((blk, k), lambda i: (i, 0))] + [
            spec
            for _ in range(4)
            for spec in (pl.BlockSpec((k, DH), lambda i: (0, 0)),
                         pl.BlockSpec((1, DH), lambda i: (0, 0)))
        ],
        out_specs=[
            pl.BlockSpec((blk, DH), lambda i: (i, 0)),
            pl.BlockSpec((blk, DH), lambda i: (i, 0)),
        ],
        out_shape=[jax.ShapeDtypeStruct((n, DH), jnp.float32)] * 2,
    )(x, w1[:, 0::2], b1[0::2].reshape(1, DH), w1[:, 1::2], b1[1::2].reshape(1, DH),
      w2[:, 0::2], b2[0::2].reshape(1, DH), w2[:, 1::2], b2[1::2].reshape(1, DH))


def _head_body(p_ref, w1_ref, b1_ref, a_ref, g_ref, be_ref, m_ref, v_ref,
               w2_ref, b2_ref, o_ref):
    ro = jnp.sum(p_ref[...], axis=0, keepdims=True)
    h = jnp.dot(ro, w1_ref[...], preferred_element_type=jnp.float32) + b1_ref[...]
    h = jnp.where(h >= 0.0, h, a_ref[...] * h)
    h = (h - m_ref[...]) / jnp.sqrt(v_ref[...] + 1e-5) * g_ref[...] + be_ref[...]
    o_ref[...] = (
        jnp.dot(h, w2_ref[...], preferred_element_type=jnp.float32) + b2_ref[...]
    )


def _head(part, fc1_W, fc1_b, fc_a, bn_g, bn_b, bn_m, bn_v, fc2_W, fc2_b):
    h2 = D // 2
    return pl.pallas_call(
        _head_body,
        out_shape=jax.ShapeDtypeStruct((1, 1), jnp.float32),
    )(
        part,
        fc1_W,
        fc1_b.reshape(1, h2),
        jnp.full((1, h2), fc_a, jnp.float32),
        bn_g.reshape(1, h2),
        bn_b.reshape(1, h2),
        bn_m.reshape(1, h2),
        bn_v.reshape(1, h2),
        fc2_W,
        fc2_b.reshape(1, 1),
    )


def _k1_body(hs, hd, e1, src_a, dst_a, attw, attb, alpha,
             wf_out, mx_out,
             src_t, dst_t, rows_sA, rows_dA, rows_eA, rows_sB, rows_dB,
             rows_eB, wf_blk, t256, attw_v, attb_v, alpha_v, mx_v,
             semA, semB):
    cid = lax.axis_index("c")
    sid = lax.axis_index("s")
    wid = sid * NC + cid
    base = wid * EPT
    pltpu.sync_copy(attw, attw_v)
    pltpu.sync_copy(attb, attb_v)
    pltpu.sync_copy(alpha, alpha_v)
    pltpu.sync_copy(src_a.at[pl.ds(base, EPT)], src_t)
    pltpu.sync_copy(dst_a.at[pl.ds(base, EPT)], dst_t)
    al = alpha_v[...]
    ab = attb_v[...]
    aw_t = [
        plsc.bitcast(attw_v[pl.ds(t * 16, 16)], jnp.bfloat16) for t in range(4)
    ]
    scat_base = lax.iota(jnp.int32, 16) * 16

    def issue(b, rows_s, rows_d, rows_e, sem):
        pltpu.async_copy(hs.at[src_t.at[pl.ds(b * BLK, BLK)]], rows_s, sem)
        pltpu.async_copy(hd.at[dst_t.at[pl.ds(b * BLK, BLK)]], rows_d, sem)
        pltpu.async_copy(e1.at[pl.ds((base + b * BLK) // 2, BLK // 2)], rows_e, sem)

    def wait(b, rows_s, rows_d, rows_e, sem):
        pltpu.make_async_copy(hs.at[src_t.at[pl.ds(b * BLK, BLK)]], rows_s, sem).wait()
        pltpu.make_async_copy(hd.at[dst_t.at[pl.ds(b * BLK, BLK)]], rows_d, sem).wait()
        pltpu.make_async_copy(
            e1.at[pl.ds((base + b * BLK) // 2, BLK // 2)], rows_e, sem).wait()

    def compute(b, rows_s, rows_d, rows_e, mx):
        for g in range(GRP):

            def pair_body(j2, carry):
                erow = g * 8 + j2
                for p_half in range(2):
                    ei = 2 * (g * 8 + j2) + p_half
                    acc = jnp.zeros((16,), jnp.float32)
                    acc2 = jnp.zeros((16,), jnp.float32)
                    for t in range(4):
                        vs = plsc.bitcast(rows_s[ei, pl.ds(t * 16, 16)], jnp.bfloat16)
                        vd = plsc.bitcast(rows_d[ei, pl.ds(t * 16, 16)], jnp.bfloat16)
                        ve = plsc.bitcast(
                            rows_e[erow, pl.ds(p_half * 64 + t * 16, 16)],
                            jnp.bfloat16)
                        w = vs + vd + ve
                        p = jnp.where(w >= 0, w, al * w)
                        q = aw_t[t] * p
                        qa, qb = plsc.unpack(
                            q, format=plsc.PackFormat.INTERLEAVED,
                            preferred_element_type=jnp.float32)
                        acc = acc + qa
                        acc2 = acc2 + qb
                    plsc.store_scatter(
                        t256, [scat_base + 2 * j2 + p_half], acc + acc2)
                return carry

            lax.fori_loop(0, 8, pair_body, 0)
            wfv = t256[pl.ds(0, 16)]
            for c in range(1, 16):
                wfv = wfv + t256[pl.ds(c * 16, 16)]
            wfv = wfv + ab
            wf_blk[pl.ds(g * 16, 16)] = wfv
            mx = jnp.maximum(mx, wfv)
        pltpu.sync_copy(wf_blk, wf_out.at[pl.ds(base + b * BLK, BLK)])
        return mx

    issue(0, rows_sA, rows_dA, rows_eA, semA)

    def pair(i, mx):
        b0 = 2 * i
        issue(b0 + 1, rows_sB, rows_dB, rows_eB, semB)
        wait(b0, rows_sA, rows_dA, rows_eA, semA)
        mx = compute(b0, rows_sA, rows_dA, rows_eA, mx)
        issue(b0 + 2, rows_sA, rows_dA, rows_eA, semA)
        wait(b0 + 1, rows_sB, rows_dB, rows_eB, semB)
        return compute(b0 + 1, rows_sB, rows_dB, rows_eB, mx)

    mx = lax.fori_loop(0, NBLK // 2, pair, jnp.full((16,), NEG, jnp.float32))
    wait(NBLK - 1, rows_sA, rows_dA, rows_eA, semA)
    mx = compute(NBLK - 1, rows_sA, rows_dA, rows_eA, mx)
    mx_v[...] = mx
    pltpu.sync_copy(mx_v, mx_out.at[wid])


def _k1(hs, hd, e1, src, dst, attw, attb16, alpha32):
    return pl.kernel(
        _k1_body,
        out_type=[
            jax.ShapeDtypeStruct((E,), jnp.float32),
            jax.ShapeDtypeStruct((NT, 16), jnp.float32),
        ],
        mesh=_mesh,
        compiler_params=pltpu.CompilerParams(needs_layout_passes=False, use_tc_tiling_on_sc=False),
        scratch_types=[
            pltpu.VMEM((EPT,), jnp.int32),
            pltpu.VMEM((EPT,), jnp.int32),
            pltpu.VMEM((BLK, DH), jnp.float32),
            pltpu.VMEM((BLK, DH), jnp.float32),
            pltpu.VMEM((BLK // 2, D), jnp.float32),
            pltpu.VMEM((BLK, DH), jnp.float32),
            pltpu.VMEM((BLK, DH), jnp.float32),
            pltpu.VMEM((BLK // 2, D), jnp.float32),
            pltpu.VMEM((BLK,), jnp.float32),
            pltpu.VMEM((256,), jnp.float32),
            pltpu.VMEM((DH,), jnp.float32),
            pltpu.VMEM((16,), jnp.float32),
            pltpu.VMEM((32,), jnp.bfloat16),
            pltpu.VMEM((16,), jnp.float32),
            pltpu.SemaphoreType.DMA,
            pltpu.SemaphoreType.DMA,
        ],
    )(hs, hd, e1, src, dst, attw, attb16, alpha32)


def _k2_body(wf, dst_a, mx_a,
             z_out, s_out,
             wf_t, z_t, idxb_v, mx_v, zero_v, s_sh, sem):
    cid = lax.axis_index("c")
    sid = lax.axis_index("s")
    wid = sid * NC + cid
    base = wid * EPT
    pltpu.sync_copy(mx_a, mx_v)
    m = jnp.full((16,), NEG, jnp.float32)
    for i in range(NT):
        m = jnp.maximum(m, mx_v[i, ...])
    mv = jnp.full((16,), jnp.max(m), jnp.float32)

    pltpu.sync_copy(wf.at[pl.ds(base, EPT)], wf_t)

    def zstep(i, carry):
        z_t[pl.ds(i * 16, 16)] = jnp.exp(wf_t[pl.ds(i * 16, 16)] - mv)
        return carry

    lax.fori_loop(0, EPT // 16, zstep, 0)
    pltpu.sync_copy(z_t, z_out.at[pl.ds(base, EPT)])

    @pl.when(sid == 0)
    def _zero():
        def zz(i, carry):
            zero_v[pl.ds(i * 16, 16)] = jnp.zeros((16,), jnp.float32)
            return carry

        lax.fori_loop(0, SPAD // 16, zz, 0)
        pltpu.sync_copy(zero_v, s_sh)

    plsc.subcore_barrier()

    def scat(b, carry):
        pltpu.sync_copy(dst_a.at[pl.ds(base + b * BLK, BLK)], idxb_v)
        pltpu.sync_copy(z_t.at[pl.ds(b * BLK, BLK)], s_sh.at[idxb_v], add=True)
        return carry

    lax.fori_loop(0, NBLK, scat, 0)
    plsc.subcore_barrier()

    @pl.when(sid == 0)
    def _flush():
        pltpu.sync_copy(s_sh, s_out.at[cid])


def _k2(wf, dst, mx):
    return pl.kernel(
        _k2_body,
        out_type=[
            jax.ShapeDtypeStruct((E,), jnp.float32),
            jax.ShapeDtypeStruct((NC, SPAD), jnp.float32),
        ],
        mesh=_mesh,
        compiler_params=pltpu.CompilerParams(needs_layout_passes=False, use_tc_tiling_on_sc=False),
        scratch_types=[
            pltpu.VMEM((EPT,), jnp.float32),
            pltpu.VMEM((EPT,), jnp.float32),
            pltpu.VMEM((BLK,), jnp.int32),
            pltpu.VMEM((NT, 16), jnp.float32),
            pltpu.VMEM((SPAD,), jnp.float32),
            pltpu.VMEM_SHARED((SPAD,), jnp.float32),
            pltpu.SemaphoreType.DMA,
        ],
    )(wf, dst, mx)


def _k3_body(z_a, src_a, dst_a, hs2, hd2, e2, s_a,
             part_out,
             src_t, dst_t, z_t, a_blk, rows_sA, rows_dA, rows_eA,
             rows_sB, rows_dB, rows_eB, gs_v, gsb_v, acc_v, semA, semB):
    cid = lax.axis_index("c")
    sid = lax.axis_index("s")
    wid = sid * NC + cid
    base = wid * EPT
    pltpu.sync_copy(s_a.at[0], gs_v)
    pltpu.sync_copy(s_a.at[1], gsb_v)
    pltpu.sync_copy(src_a.at[pl.ds(base, EPT)], src_t)
    pltpu.sync_copy(dst_a.at[pl.ds(base, EPT)], dst_t)
    pltpu.sync_copy(z_a.at[pl.ds(base, EPT)], z_t)

    def addg(i, carry):
        gs_v[pl.ds(i * 16, 16)] = gs_v[pl.ds(i * 16, 16)] + gsb_v[pl.ds(i * 16, 16)]
        return carry

    lax.fori_loop(0, SPAD // 16, addg, 0)

    def issue(b, rows_s, rows_d, rows_e, sem):
        pltpu.async_copy(hs2.at[src_t.at[pl.ds(b * BLK, BLK)]], rows_s, sem)
        pltpu.async_copy(hd2.at[dst_t.at[pl.ds(b * BLK, BLK)]], rows_d, sem)
        pltpu.async_copy(e2.at[pl.ds((base + b * BLK) // 2, BLK // 2)], rows_e, sem)

    def wait(b, rows_s, rows_d, rows_e, sem):
        pltpu.make_async_copy(hs2.at[src_t.at[pl.ds(b * BLK, BLK)]], rows_s, sem).wait()
        pltpu.make_async_copy(hd2.at[dst_t.at[pl.ds(b * BLK, BLK)]], rows_d, sem).wait()
        pltpu.make_async_copy(
            e2.at[pl.ds((base + b * BLK) // 2, BLK // 2)], rows_e, sem).wait()

    def compute(b, rows_s, rows_d, rows_e, accs):
        for g in range(GRP):
            zv = z_t[pl.ds(b * BLK + g * 16, 16)]
            dv = dst_t[pl.ds(b * BLK + g * 16, 16)]
            sv = plsc.load_gather(gs_v, [dv])
            a_blk[pl.ds(g * 16, 16)] = zv / sv

        def pair_body(j2, accs2):
            cur = list(accs2)
            for p_half in range(2):
                ei = 2 * j2 + p_half
                aj = plsc.load_gather(a_blk, [jnp.full((16,), ei, jnp.int32)])
                for t in range(4):
                    vs = plsc.bitcast(rows_s[ei, pl.ds(t * 16, 16)], jnp.bfloat16)
                    vd = plsc.bitcast(rows_d[ei, pl.ds(t * 16, 16)], jnp.bfloat16)
                    ve = plsc.bitcast(
                        rows_e[j2, pl.ds(p_half * 64 + t * 16, 16)], jnp.bfloat16)
                    m = vs * vd * ve
                    ma, mb = plsc.unpack(
                        m, format=plsc.PackFormat.INTERLEAVED,
                        preferred_element_type=jnp.float32)
                    cur[2 * t] = cur[2 * t] + aj * ma
                    cur[2 * t + 1] = cur[2 * t + 1] + aj * mb
            return tuple(cur)

        return lax.fori_loop(0, BLK // 2, pair_body, accs)

    issue(0, rows_sA, rows_dA, rows_eA, semA)

    def pair(i, accs):
        b0 = 2 * i
        issue(b0 + 1, rows_sB, rows_dB, rows_eB, semB)
        wait(b0, rows_sA, rows_dA, rows_eA, semA)
        accs = compute(b0, rows_sA, rows_dA, rows_eA, accs)
        issue(b0 + 2, rows_sA, rows_dA, rows_eA, semA)
        wait(b0 + 1, rows_sB, rows_dB, rows_eB, semB)
        return compute(b0 + 1, rows_sB, rows_dB, rows_eB, accs)

    accs = lax.fori_loop(
        0, NBLK // 2, pair,
        tuple(jnp.zeros((16,), jnp.float32) for _ in range(8)),
    )
    wait(NBLK - 1, rows_sA, rows_dA, rows_eA, semA)
    accs = compute(NBLK - 1, rows_sA, rows_dA, rows_eA, accs)
    for t in range(8):
        acc_v[pl.ds(t * 16, 16)] = accs[t]
    pltpu.sync_copy(acc_v, part_out.at[wid])


def _k3(z, src, dst, hs2, hd2, e2, s_all):
    return pl.kernel(
        _k3_body,
        out_type=jax.ShapeDtypeStruct((NT, D), jnp.float32),
        mesh=_mesh,
        compiler_params=pltpu.CompilerParams(needs_layout_passes=False, use_tc_tiling_on_sc=False),
        scratch_types=[
            pltpu.VMEM((EPT,), jnp.int32),
            pltpu.VMEM((EPT,), jnp.int32),
            pltpu.VMEM((EPT,), jnp.float32),
            pltpu.VMEM((BLK,), jnp.float32),
            pltpu.VMEM((BLK, DH), jnp.float32),
            pltpu.VMEM((BLK, DH), jnp.float32),
            pltpu.VMEM((BLK // 2, D), jnp.float32),
            pltpu.VMEM((BLK, DH), jnp.float32),
            pltpu.VMEM((BLK, DH), jnp.float32),
            pltpu.VMEM((BLK // 2, D), jnp.float32),
            pltpu.VMEM((SPAD,), jnp.float32),
            pltpu.VMEM((SPAD,), jnp.float32),
            pltpu.VMEM((D,), jnp.float32),
            pltpu.SemaphoreType.DMA,
            pltpu.SemaphoreType.DMA,
        ],
    )(z, src, dst, hs2, hd2, e2, s_all)


def kernel(x_lig, x_poc, edge_feat, edge_index,
           prj_src_W, prj_src_b, prj_dst_W, prj_dst_b, prj_edge_W, prj_edge_b,
           w_src_W, w_src_b, w_dst_W, w_dst_b, w_edge_W, w_edge_b,
           att_a, att_W, att_b,
           fc1_W, fc1_b, fc_a, bn_g, bn_b, bn_m, bn_v, fc2_W, fc2_b):
    src = edge_index[0]
    dst = edge_index[1]

    hs, hs2 = _proj2(x_lig, prj_src_W, prj_src_b, w_src_W, w_src_b, 2000)
    hd, hd2 = _proj2(x_poc, prj_dst_W, prj_dst_b, w_dst_W, w_dst_b, 2000)
    ef_e = edge_feat[0::2]
    ef_o = edge_feat[1::2]
    e1 = _eproj(ef_e, ef_o, prj_edge_W, prj_edge_b, 2000)
    e2 = _eproj(ef_e, ef_o, w_edge_W, w_edge_b, 2000)

    aw = att_W[:, 0]
    ev = jax.lax.bitcast_convert_type(
        aw[0::2].astype(jnp.bfloat16), jnp.uint16).astype(jnp.uint32)
    od = jax.lax.bitcast_convert_type(
        aw[1::2].astype(jnp.bfloat16), jnp.uint16).astype(jnp.uint32)
    attwp = jax.lax.bitcast_convert_type(ev | (od << 16), jnp.float32)
    attb16 = jnp.full((16,), att_b[0], jnp.float32)
    alpha32 = jnp.full((32,), att_a, jnp.bfloat16)
    wf, mx = _k1(hs, hd, e1, src, dst, attwp, attb16, alpha32)
    z, s_all = _k2(wf, dst, mx)
    part = _k3(z, src, dst, hs2, hd2, e2, s_all)
    perm = []
    for t in range(4):
        perm += [32 * t + 2 * i for i in range(16)]
        perm += [32 * t + 2 * i + 1 for i in range(16)]
    fc1_Wp = fc1_W[jnp.array(perm, jnp.int32), :]
    return _head(part, fc1_Wp, fc1_b, fc_a, bn_g, bn_b, bn_m, bn_v, fc2_W, fc2_b)


# speedup vs baseline: 1.8393x; 1.8393x over previous
import functools

import jax
import jax.numpy as jnp
from jax import lax
from jax.experimental import pallas as pl
from jax.experimental.pallas import tpu as pltpu
from jax.experimental.pallas import tpu_sc as plsc

N_NODE = 10000
E = 320000
D = 128
NC = 2
NS = 16
NT = NC * NS
EPT = E // NT
BLK = 80
NBLK = EPT // BLK
GRP = BLK // 16
SPAD = 10240
NEG = -3.0e38

_mesh = plsc.VectorSubcoreMesh(core_axis_name="c", subcore_axis_name="s")


def _pack_pairs(yev, yod):
    ev = jax.lax.bitcast_convert_type(yev.astype(jnp.bfloat16), jnp.uint16)
    od = jax.lax.bitcast_convert_type(yod.astype(jnp.bfloat16), jnp.uint16)
    word = ev.astype(jnp.uint32) | (od.astype(jnp.uint32) << 16)
    return jax.lax.bitcast_convert_type(word, jnp.float32)


def _proj2_body(x_ref, w1e_ref, b1e_ref, w1o_ref, b1o_ref,
                w2e_ref, b2e_ref, w2o_ref, b2o_ref, o1_ref, o2_ref):
    x = x_ref[...]
    o1_ref[...] = _pack_pairs(
        jnp.dot(x, w1e_ref[...], preferred_element_type=jnp.float32) + b1e_ref[...],
        jnp.dot(x, w1o_ref[...], preferred_element_type=jnp.float32) + b1o_ref[...],
    )
    o2_ref[...] = _pack_pairs(
        jnp.dot(x, w2e_ref[...], preferred_element_type=jnp.float32) + b2e_ref[...],
        jnp.dot(x, w2o_ref[...], preferred_element_type=jnp.float32) + b2o_ref[...],
    )


DH = D // 2


def _eproj_body(x2_ref, we_ref, be_ref, wo_ref, bo_ref, o_ref):
    xe = x2_ref[:, 0:16]
    xo = x2_ref[:, 16:32]
    pe = _pack_pairs(
        jnp.dot(xe, we_ref[...], preferred_element_type=jnp.float32) + be_ref[...],
        jnp.dot(xe, wo_ref[...], preferred_element_type=jnp.float32) + bo_ref[...],
    )
    po = _pack_pairs(
        jnp.dot(xo, we_ref[...], preferred_element_type=jnp.float32) + be_ref[...],
        jnp.dot(xo, wo_ref[...], preferred_element_type=jnp.float32) + bo_ref[...],
    )
    o_ref[...] = jnp.concatenate([pe, po], axis=1)


def _eproj(x2, w, b, blk):
    n2 = x2.shape[0]
    return pl.pallas_call(
        _eproj_body,
        grid=(n2 // blk,),
        in_specs=[
            pl.BlockSpec((blk, 32), lambda i: (i, 0)),
            pl.BlockSpec((16, DH), lambda i: (0, 0)),
            pl.BlockSpec((1, DH), lambda i: (0, 0)),
            pl.BlockSpec((16, DH), lambda i: (0, 0)),
            pl.BlockSpec((1, DH), lambda i: (0, 0)),
        ],
        out_specs=pl.BlockSpec((blk, D), lambda i: (i, 0)),
        out_shape=jax.ShapeDtypeStruct((n2, D), jnp.float32),
    )(x2, w[:, 0::2], b[0::2].reshape(1, DH), w[:, 1::2], b[1::2].reshape(1, DH))


def _proj2(x, w1, b1, w2, b2, blk):
    n, k = x.shape
    grid = n // blk
    return pl.pallas_call(
        _proj2_body,
        grid=(grid,),
        in_specs=[pl.BlockSpec((blk, k), lambda i: (i, 0))] + [
            spec
            for _ in range(4)
            for spec in (pl.BlockSpec((k, DH), lambda i: (0, 0)),
                         pl.BlockSpec((1, DH), lambda i: (0, 0)))
        ],
        out_specs=[
            pl.BlockSpec((blk, DH), lambda i: (i, 0)),
            pl.BlockSpec((blk, DH), lambda i: (i, 0)),
        ],
        out_shape=[jax.ShapeDtypeStruct((n, DH), jnp.float32)] * 2,
    )(x, w1[:, 0::2], b1[0::2].reshape(1, DH), w1[:, 1::2], b1[1::2].reshape(1, DH),
      w2[:, 0::2], b2[0::2].reshape(1, DH), w2[:, 1::2], b2[1::2].reshape(1, DH))


def _head_body(p_ref, w1_ref, b1_ref, a_ref, g_ref, be_ref, m_ref, v_ref,
               w2_ref, b2_ref, o_ref):
    ro = jnp.sum(p_ref[...], axis=0, keepdims=True)
    h = jnp.dot(ro, w1_ref[...], preferred_element_type=jnp.float32) + b1_ref[...]
    h = jnp.where(h >= 0.0, h, a_ref[...] * h)
    h = (h - m_ref[...]) / jnp.sqrt(v_ref[...] + 1e-5) * g_ref[...] + be_ref[...]
    o_ref[...] = (
        jnp.dot(h, w2_ref[...], preferred_element_type=jnp.float32) + b2_ref[...]
    )


def _head(part, fc1_W, fc1_b, fc_a, bn_g, bn_b, bn_m, bn_v, fc2_W, fc2_b):
    h2 = D // 2
    return pl.pallas_call(
        _head_body,
        out_shape=jax.ShapeDtypeStruct((1, 1), jnp.float32),
    )(
        part,
        fc1_W,
        fc1_b.reshape(1, h2),
        jnp.full((1, h2), fc_a, jnp.float32),
        bn_g.reshape(1, h2),
        bn_b.reshape(1, h2),
        bn_m.reshape(1, h2),
        bn_v.reshape(1, h2),
        fc2_W,
        fc2_b.reshape(1, 1),
    )


def _k1_body(hs, hd, e1, src_a, dst_a, attw, attb, alpha,
             wf_out, mx_out,
             src_t, dst_t, rows_sA, rows_dA, rows_eA, rows_sB, rows_dB,
             rows_eB, wf_blk, t256, attw_v, attb_v, alpha_v, mx_v,
             semA, semB):
    cid = lax.axis_index("c")
    sid = lax.axis_index("s")
    wid = sid * NC + cid
    base = wid * EPT
    pltpu.sync_copy(attw, attw_v)
    pltpu.sync_copy(attb, attb_v)
    pltpu.sync_copy(alpha, alpha_v)
    pltpu.sync_copy(src_a.at[pl.ds(base, EPT)], src_t)
    pltpu.sync_copy(dst_a.at[pl.ds(base, EPT)], dst_t)
    al = alpha_v[...]
    ab = attb_v[...]
    aw_t = [
        plsc.bitcast(attw_v[pl.ds(t * 16, 16)], jnp.bfloat16) for t in range(4)
    ]
    scat_base = lax.iota(jnp.int32, 16) * 16

    def issue(b, rows_s, rows_d, rows_e, sem):
        pltpu.async_copy(hs.at[src_t.at[pl.ds(b * BLK, BLK)]], rows_s, sem)
        pltpu.async_copy(hd.at[dst_t.at[pl.ds(b * BLK, BLK)]], rows_d, sem)
        pltpu.async_copy(e1.at[pl.ds((base + b * BLK) // 2, BLK // 2)], rows_e, sem)

    def wait(b, rows_s, rows_d, rows_e, sem):
        pltpu.make_async_copy(hs.at[src_t.at[pl.ds(b * BLK, BLK)]], rows_s, sem).wait()
        pltpu.make_async_copy(hd.at[dst_t.at[pl.ds(b * BLK, BLK)]], rows_d, sem).wait()
        pltpu.make_async_copy(
            e1.at[pl.ds((base + b * BLK) // 2, BLK // 2)], rows_e, sem).wait()

    def compute(b, rows_s, rows_d, rows_e, mx):
        for g in range(GRP):

            def pair_body(j2, carry):
                erow = g * 8 + j2
                for p_half in range(2):
                    ei = 2 * (g * 8 + j2) + p_half
                    acc = jnp.zeros((16,), jnp.float32)
                    acc2 = jnp.zeros((16,), jnp.float32)
                    for t in range(4):
                        vs = plsc.bitcast(rows_s[ei, pl.ds(t * 16, 16)], jnp.bfloat16)
                        vd = plsc.bitcast(rows_d[ei, pl.ds(t * 16, 16)], jnp.bfloat16)
                        ve = plsc.bitcast(
                            rows_e[erow, pl.ds(p_half * 64 + t * 16, 16)],
                            jnp.bfloat16)
                        w = vs + vd + ve
                        p = jnp.where(w >= 0, w, al * w)
                        q = aw_t[t] * p
                        qa, qb = plsc.unpack(
                            q, format=plsc.PackFormat.INTERLEAVED,
                            preferred_element_type=jnp.float32)
                        acc = acc + qa
                        acc2 = acc2 + qb
                    plsc.store_scatter(
                        t256, [scat_base + 2 * j2 + p_half], acc + acc2)
                return carry

            lax.fori_loop(0, 8, pair_body, 0)
            wfv = t256[pl.ds(0, 16)]
            for c in range(1, 16):
                wfv = wfv + t256[pl.ds(c * 16, 16)]
            wfv = wfv + ab
            wf_blk[pl.ds(g * 16, 16)] = wfv
            mx = jnp.maximum(mx, wfv)
        pltpu.sync_copy(wf_blk, wf_out.at[pl.ds(base + b * BLK, BLK)])
        return mx

    issue(0, rows_sA, rows_dA, rows_eA, semA)

    def pair(i, mx):
        b0 = 2 * i
        issue(b0 + 1, rows_sB, rows_dB, rows_eB, semB)
        wait(b0, rows_sA, rows_dA, rows_eA, semA)
        mx = compute(b0, rows_sA, rows_dA, rows_eA, mx)
        issue(b0 + 2, rows_sA, rows_dA, rows_eA, semA)
        wait(b0 + 1, rows_sB, rows_dB, rows_eB, semB)
        return compute(b0 + 1, rows_sB, rows_dB, rows_eB, mx)

    mx = lax.fori_loop(0, NBLK // 2, pair, jnp.full((16,), NEG, jnp.float32))
    wait(NBLK - 1, rows_sA, rows_dA, rows_eA, semA)
    mx = compute(NBLK - 1, rows_sA, rows_dA, rows_eA, mx)
    mx_v[...] = mx
    pltpu.sync_copy(mx_v, mx_out.at[wid])


def _k1(hs, hd, e1, src, dst, attw, attb16, alpha32):
    return pl.kernel(
        _k1_body,
        out_type=[
            jax.ShapeDtypeStruct((E,), jnp.float32),
            jax.ShapeDtypeStruct((NT, 16), jnp.float32),
        ],
        mesh=_mesh,
        compiler_params=pltpu.CompilerParams(needs_layout_passes=False, use_tc_tiling_on_sc=False),
        scratch_types=[
            pltpu.VMEM((EPT,), jnp.int32),
            pltpu.VMEM((EPT,), jnp.int32),
            pltpu.VMEM((BLK, DH), jnp.float32),
            pltpu.VMEM((BLK, DH), jnp.float32),
            pltpu.VMEM((BLK // 2, D), jnp.float32),
            pltpu.VMEM((BLK, DH), jnp.float32),
            pltpu.VMEM((BLK, DH), jnp.float32),
            pltpu.VMEM((BLK // 2, D), jnp.float32),
            pltpu.VMEM((BLK,), jnp.float32),
            pltpu.VMEM((256,), jnp.float32),
            pltpu.VMEM((DH,), jnp.float32),
            pltpu.VMEM((16,), jnp.float32),
            pltpu.VMEM((32,), jnp.bfloat16),
            pltpu.VMEM((16,), jnp.float32),
            pltpu.SemaphoreType.DMA,
            pltpu.SemaphoreType.DMA,
        ],
    )(hs, hd, e1, src, dst, attw, attb16, alpha32)


def _k2_body(wf, dst_a, mx_a,
             z_out, s_out,
             wf_t, z_t, idxb_v, mx_v, zero_v, s_sh, sem):
    cid = lax.axis_index("c")
    sid = lax.axis_index("s")
    wid = sid * NC + cid
    base = wid * EPT
    pltpu.sync_copy(mx_a, mx_v)
    m = jnp.full((16,), NEG, jnp.float32)
    for i in range(NT):
        m = jnp.maximum(m, mx_v[i, ...])
    mv = jnp.full((16,), jnp.max(m), jnp.float32)

    pltpu.sync_copy(wf.at[pl.ds(base, EPT)], wf_t)

    def zstep(i, carry):
        z_t[pl.ds(i * 16, 16)] = jnp.exp(wf_t[pl.ds(i * 16, 16)] - mv)
        return carry

    lax.fori_loop(0, EPT // 16, zstep, 0)
    pltpu.sync_copy(z_t, z_out.at[pl.ds(base, EPT)])

    @pl.when(sid == 0)
    def _zero():
        def zz(i, carry):
            zero_v[pl.ds(i * 16, 16)] = jnp.zeros((16,), jnp.float32)
            return carry

        lax.fori_loop(0, SPAD // 16, zz, 0)
        pltpu.sync_copy(zero_v, s_sh)

    plsc.subcore_barrier()

    def scat(b, carry):
        pltpu.sync_copy(dst_a.at[pl.ds(base + b * BLK, BLK)], idxb_v)
        pltpu.sync_copy(z_t.at[pl.ds(b * BLK, BLK)], s_sh.at[idxb_v], add=True)
        return carry

    lax.fori_loop(0, NBLK, scat, 0)
    plsc.subcore_barrier()

    @pl.when(sid == 0)
    def _flush():
        pltpu.sync_copy(s_sh, s_out.at[cid])


def _k2(wf, dst, mx):
    return pl.kernel(
        _k2_body,
        out_type=[
            jax.ShapeDtypeStruct((E,), jnp.float32),
            jax.ShapeDtypeStruct((NC, SPAD), jnp.float32),
        ],
        mesh=_mesh,
        compiler_params=pltpu.CompilerParams(needs_layout_passes=False, use_tc_tiling_on_sc=False),
        scratch_types=[
            pltpu.VMEM((EPT,), jnp.float32),
            pltpu.VMEM((EPT,), jnp.float32),
            pltpu.VMEM((BLK,), jnp.int32),
            pltpu.VMEM((NT, 16), jnp.float32),
            pltpu.VMEM((SPAD,), jnp.float32),
            pltpu.VMEM_SHARED((SPAD,), jnp.float32),
            pltpu.SemaphoreType.DMA,
        ],
    )(wf, dst, mx)


def _k3_body(z_a, src_a, dst_a, hs2, hd2, e2, s_a,
             part_out,
             src_t, dst_t, z_t, a_blk, rows_sA, rows_dA, rows_eA,
             rows_sB, rows_dB, rows_eB, gs_v, gsb_v, acc_v, semA, semB):
    cid = lax.axis_index("c")
    sid = lax.axis_index("s")
    wid = sid * NC + cid
    base = wid * EPT
    pltpu.sync_copy(s_a.at[0], gs_v)
    pltpu.sync_copy(s_a.at[1], gsb_v)
    pltpu.sync_copy(src_a.at[pl.ds(base, EPT)], src_t)
    pltpu.sync_copy(dst_a.at[pl.ds(base, EPT)], dst_t)
    pltpu.sync_copy(z_a.at[pl.ds(base, EPT)], z_t)

    def addg(i, carry):
        gs_v[pl.ds(i * 16, 16)] = gs_v[pl.ds(i * 16, 16)] + gsb_v[pl.ds(i * 16, 16)]
        return carry

    lax.fori_loop(0, SPAD // 16, addg, 0)

    def issue(b, rows_s, rows_d, rows_e, sem):
        pltpu.async_copy(hs2.at[src_t.at[pl.ds(b * BLK, BLK)]], rows_s, sem)
        pltpu.async_copy(hd2.at[dst_t.at[pl.ds(b * BLK, BLK)]], rows_d, sem)
        pltpu.async_copy(e2.at[pl.ds((base + b * BLK) // 2, BLK // 2)], rows_e, sem)

    def wait(b, rows_s, rows_d, rows_e, sem):
        pltpu.make_async_copy(hs2.at[src_t.at[pl.ds(b * BLK, BLK)]], rows_s, sem).wait()
        pltpu.make_async_copy(hd2.at[dst_t.at[pl.ds(b * BLK, BLK)]], rows_d, sem).wait()
        pltpu.make_async_copy(
            e2.at[pl.ds((base + b * BLK) // 2, BLK // 2)], rows_e, sem).wait()

    def compute(b, rows_s, rows_d, rows_e, accs):
        for g in range(GRP):
            zv = z_t[pl.ds(b * BLK + g * 16, 16)]
            dv = dst_t[pl.ds(b * BLK + g * 16, 16)]
            sv = plsc.load_gather(gs_v, [dv])
            a_blk[pl.ds(g * 16, 16)] = zv / sv

        def pair_body(j2, accs2):
            cur = list(accs2)
            for p_half in range(2):
                ei = 2 * j2 + p_half
                aj = plsc.load_gather(a_blk, [jnp.full((16,), ei, jnp.int32)])
                for t in range(4):
                    vs = plsc.bitcast(rows_s[ei, pl.ds(t * 16, 16)], jnp.bfloat16)
                    vd = plsc.bitcast(rows_d[ei, pl.ds(t * 16, 16)], jnp.bfloat16)
                    ve = plsc.bitcast(
                        rows_e[j2, pl.ds(p_half * 64 + t * 16, 16)], jnp.bfloat16)
                    m = vs * vd * ve
                    ma, mb = plsc.unpack(
                        m, format=plsc.PackFormat.INTERLEAVED,
                        preferred_element_type=jnp.float32)
                    cur[2 * t] = cur[2 * t] + aj * ma
                    cur[2 * t + 1] = cur[2 * t + 1] + aj * mb
            return tuple(cur)

        return lax.fori_loop(0, BLK // 2, pair_body, accs)

    issue(0, rows_sA, rows_dA, rows_eA, semA)

    def pair(i, accs):
        b0 = 2 * i
        issue(b0 + 1, rows_sB, rows_dB, rows_eB, semB)
        wait(b0, rows_sA, rows_dA, rows_eA, semA)
        accs = compute(b0, rows_sA, rows_dA, rows_eA, accs)
        issue(b0 + 2, rows_sA, rows_dA, rows_eA, semA)
        wait(b0 + 1, rows_sB, rows_dB, rows_eB, semB)
        return compute(b0 + 1, rows_sB, rows_dB, rows_eB, accs)

    accs = lax.fori_loop(
        0, NBLK // 2, pair,
        tuple(jnp.zeros((16,), jnp.float32) for _ in range(8)),
    )
    wait(NBLK - 1, rows_sA, rows_dA, rows_eA, semA)
    accs = compute(NBLK - 1, rows_sA, rows_dA, rows_eA, accs)
    for t in range(8):
        acc_v[pl.ds(t * 16, 16)] = accs[t]
    pltpu.sync_copy(acc_v, part_out.at[wid])


def _k3(z, src, dst, hs2, hd2, e2, s_all):
    return pl.kernel(
        _k3_body,
        out_type=jax.ShapeDtypeStruct((NT, D), jnp.float32),
        mesh=_mesh,
        compiler_params=pltpu.CompilerParams(needs_layout_passes=False, use_tc_tiling_on_sc=False),
        scratch_types=[
            pltpu.VMEM((EPT,), jnp.int32),
            pltpu.VMEM((EPT,), jnp.int32),
            pltpu.VMEM((EPT,), jnp.float32),
            pltpu.VMEM((BLK,), jnp.float32),
            pltpu.VMEM((BLK, DH), jnp.float32),
            pltpu.VMEM((BLK, DH), jnp.float32),
            pltpu.VMEM((BLK // 2, D), jnp.float32),
            pltpu.VMEM((BLK, DH), jnp.float32),
            pltpu.VMEM((BLK, DH), jnp.float32),
            pltpu.VMEM((BLK // 2, D), jnp.float32),
            pltpu.VMEM((SPAD,), jnp.float32),
            pltpu.VMEM((SPAD,), jnp.float32),
            pltpu.VMEM((D,), jnp.float32),
            pltpu.SemaphoreType.DMA,
            pltpu.SemaphoreType.DMA,
        ],
    )(z, src, dst, hs2, hd2, e2, s_all)


def kernel(x_lig, x_poc, edge_feat, edge_index,
           prj_src_W, prj_src_b, prj_dst_W, prj_dst_b, prj_edge_W, prj_edge_b,
           w_src_W, w_src_b, w_dst_W, w_dst_b, w_edge_W, w_edge_b,
           att_a, att_W, att_b,
           fc1_W, fc1_b, fc_a, bn_g, bn_b, bn_m, bn_v, fc2_W, fc2_b):
    src = edge_index[0]
    dst = edge_index[1]

    hs, hs2 = _proj2(x_lig, prj_src_W, prj_src_b, w_src_W, w_src_b, 2000)
    hd, hd2 = _proj2(x_poc, prj_dst_W, prj_dst_b, w_dst_W, w_dst_b, 2000)
    ef2 = edge_feat.reshape(E // 2, 32)
    e1 = _eproj(ef2, prj_edge_W, prj_edge_b, 2000)
    e2 = _eproj(ef2, w_edge_W, w_edge_b, 2000)

    aw = att_W[:, 0]
    ev = jax.lax.bitcast_convert_type(
        aw[0::2].astype(jnp.bfloat16), jnp.uint16).astype(jnp.uint32)
    od = jax.lax.bitcast_convert_type(
        aw[1::2].astype(jnp.bfloat16), jnp.uint16).astype(jnp.uint32)
    attwp = jax.lax.bitcast_convert_type(ev | (od << 16), jnp.float32)
    attb16 = jnp.full((16,), att_b[0], jnp.float32)
    alpha32 = jnp.full((32,), att_a, jnp.bfloat16)
    wf, mx = _k1(hs, hd, e1, src, dst, attwp, attb16, alpha32)
    z, s_all = _k2(wf, dst, mx)
    part = _k3(z, src, dst, hs2, hd2, e2, s_all)
    perm = []
    for t in range(4):
        perm += [32 * t + 2 * i for i in range(16)]
        perm += [32 * t + 2 * i + 1 for i in range(16)]
    fc1_Wp = fc1_W[jnp.array(perm, jnp.int32), :]
    return _head(part, fc1_Wp, fc1_b, fc_a, bn_g, bn_b, bn_m, bn_v, fc2_W, fc2_b)
